# baseline (speedup 1.0000x reference)
import functools

import jax
import jax.numpy as jnp
from jax import lax
from jax.experimental import pallas as pl
from jax.experimental.pallas import tpu as pltpu
from jax.experimental.pallas import tpu_sc as plsc

N = 10000
NP = 10240
E = 320000
EP = 327680
H = 8
D = 16
DIN = 128
RW = 128
HH = 4
NT = 16
RPT = NP // NT
EPT = EP // NT
CHA = 512
NCHA = EPT // CHA
CHB = 64
SCE = 512
SCROWS = SCE // CHB
ERT = EPT // CHB
NSUP = EPT // SCE
NM = NSUP // 2
BLK = 512
NBLK = NP // BLK


def _tc_proj_body(x_ref, wt_ref, ssrc_ref, stgt_ref, pe_ref, *score_refs):
    x = x_ref[:]
    p = jnp.dot(x, wt_ref[:], preferred_element_type=jnp.float32)
    ss = lax.dot_general(ssrc_ref[:], p, (((1,), (1,)), ((), ())),
                         preferred_element_type=jnp.float32)
    ts = lax.dot_general(stgt_ref[:], p, (((1,), (1,)), ((), ())),
                         preferred_element_type=jnp.float32)
    pe_ref[...] = p
    for g in range(H):
        score_refs[g][...] = ss[g]
        score_refs[H + g][...] = ts[g]


def _sc_score_body(src_hbm, tgt_hbm, *rest):
    score_hbm = rest[:2 * H]
    w0_hbm, w1_hbm = rest[2 * H:2 * H + 2]
    stab, ttab, srcidx, tgtidx, wout = rest[2 * H + 2:]
    c = lax.axis_index("c")
    t = lax.axis_index("s")
    iota16 = lax.iota(jnp.int32, 16)

    @pl.when(c == 0)
    def _():
        for h in range(HH):
            pltpu.sync_copy(score_hbm[h], stab.at[pl.ds(h * NP, NP)])
            pltpu.sync_copy(score_hbm[H + h], ttab.at[pl.ds(h * NP, NP)])

    @pl.when(c == 1)
    def _():
        for h in range(HH):
            pltpu.sync_copy(score_hbm[HH + h], stab.at[pl.ds(h * NP, NP)])
            pltpu.sync_copy(score_hbm[H + HH + h], ttab.at[pl.ds(h * NP, NP)])

    e0 = t * EPT

    def chunk(i, carry):
        base = e0 + i * CHA
        pltpu.sync_copy(src_hbm.at[pl.ds(base, CHA)], srcidx)
        pltpu.sync_copy(tgt_hbm.at[pl.ds(base, CHA)], tgtidx)
        for v in range(CHA // 16):
            sv = srcidx[pl.ds(v * 16, 16)]
            tv = tgtidx[pl.ds(v * 16, 16)]
            for h in range(HH):
                ssv = plsc.load_gather(stab, [sv + h * NP])
                tsv = plsc.load_gather(ttab, [tv + h * NP])
                s = ssv + tsv
                w = jnp.exp(jnp.where(s >= 0, s, s * 0.2))
                plsc.store_scatter(wout, [(iota16 + v * 16) * HH + h], w)

        @pl.when(c == 0)
        def _():
            pltpu.sync_copy(wout, w0_hbm.at[pl.ds(base * HH, CHA * HH)])

        @pl.when(c == 1)
        def _():
            pltpu.sync_copy(wout, w1_hbm.at[pl.ds(base * HH, CHA * HH)])
        return carry

    lax.fori_loop(0, NCHA, chunk, 0)


def _sc_agg_body(pe_hbm, w0_hbm, w1_hbm, src2_hbm, tgt2_hbm, acc_hbm,
                 acc, srcA, tgtA, wscA, srcB, tgtB, wscB,
                 rb0, rb1, ob0, ob1,
                 gs0, gs1, ss0, ss1, isA, isB, wsA, wsB):
    c = lax.axis_index("c")
    t = lax.axis_index("s")
    r0 = t * RPT
    row0 = t * ERT
    iota16 = lax.iota(jnp.int32, 16)
    zeros16 = jnp.zeros((16,), jnp.float32)
    rbs = (rb0, rb1)
    obs = (ob0, ob1)
    gss = (gs0, gs1)
    sss = (ss0, ss1)
    sets = ((srcA, tgtA, wscA, isA, wsA), (srcB, tgtB, wscB, isB, wsB))
    coff = c * 64

    def zrow(j, carry):
        jv = jnp.full((16,), j, jnp.int32)
        for ob in obs:
            for kk in range(RW // 16):
                plsc.store_scatter(ob, [jv, iota16 + kk * 16], zeros16)
        return carry
    lax.fori_loop(0, CHB, zrow, 0)
    for k in range(RPT // CHB):
        pltpu.sync_copy(obs[0], acc.at[pl.ds(r0 + k * CHB, CHB)])
    plsc.subcore_barrier()

    def issue_pref(g, bufset):
        sbuf, tbuf, wbuf, isem, wsem = bufset
        rr = row0 + g * SCROWS
        pltpu.async_copy(src2_hbm.at[pl.ds(rr, SCROWS)], sbuf, isem)
        pltpu.async_copy(tgt2_hbm.at[pl.ds(rr, SCROWS)], tbuf, isem)
        eb = (t * EPT + g * SCE) * HH

        @pl.when(c == 0)
        def _():
            pltpu.async_copy(w0_hbm.at[pl.ds(eb, SCE * HH)],
                             wbuf.at[pl.ds(0, SCE * HH)], wsem)

        @pl.when(c == 1)
        def _():
            pltpu.async_copy(w1_hbm.at[pl.ds(eb, SCE * HH)],
                             wbuf.at[pl.ds(0, SCE * HH)], wsem)

    def wait_idx(bufset):
        sbuf, tbuf, wbuf, isem, wsem = bufset
        pltpu.make_async_copy(src2_hbm.at[pl.ds(row0, SCROWS)], sbuf, isem).wait()
        pltpu.make_async_copy(tgt2_hbm.at[pl.ds(row0, SCROWS)], tbuf, isem).wait()

    def wait_w(bufset):
        sbuf, tbuf, wbuf, isem, wsem = bufset

        @pl.when(c == 0)
        def _():
            pltpu.make_async_copy(w0_hbm.at[pl.ds(0, SCE * HH)],
                                  wbuf.at[pl.ds(0, SCE * HH)], wsem).wait()

        @pl.when(c == 1)
        def _():
            pltpu.make_async_copy(w1_hbm.at[pl.ds(0, SCE * HH)],
                                  wbuf.at[pl.ds(0, SCE * HH)], wsem).wait()

    def issue_gather(srcref, p):
        pltpu.async_copy(pe_hbm.at[srcref], rbs[p], gss[p])

    def wait_gather(srcref, p):
        pltpu.make_async_copy(pe_hbm.at[srcref], rbs[p], gss[p]).wait()

    def issue_scatter(tgtref, p):
        pltpu.async_copy(obs[p], acc.at[tgtref], sss[p], add=True)

    def wait_scatter(tgtref, p):
        pltpu.make_async_copy(obs[p], acc.at[tgtref], sss[p]).wait()

    def compute(bufset, k, p):
        sbuf, tbuf, wbuf, isem, wsem = bufset
        ob = obs[p]
        rb = rbs[p]

        def make_jbody(off):
            def jbody(jq, carry):
                for q in range(4):
                    j = jq * 4 + q
                    wv = wbuf[pl.ds(k * CHB * HH + j * HH, 16)]
                    ob[j, pl.ds(64, 16)] = jnp.where(iota16 < HH, wv, 0.0)
                    for h in range(HH):
                        w = wv[h]
                        r = rb[j, pl.ds(off + h * D, D)]
                        ob[j, pl.ds(h * D, D)] = r * w
                return carry
            return jbody

        @pl.when(c == 0)
        def _():
            lax.fori_loop(0, CHB // 4, make_jbody(0), 0)

        @pl.when(c == 1)
        def _():
            lax.fori_loop(0, CHB // 4, make_jbody(64), 0)

    def do_super(m, cur, nxt, is_a):
        wait_w(cur)
        for k in range(SCROWS):
            p = k % 2
            if k == 2:
                if is_a:
                    issue_pref(2 * m + 1, nxt)
                else:
                    @pl.when(m < NM - 1)
                    def _():
                        issue_pref(2 * m + 2, nxt)
            if k == SCROWS - 1:
                if is_a:
                    wait_idx(nxt)
                    issue_gather(nxt[0].at[0], 0)
                else:
                    @pl.when(m < NM - 1)
                    def _():
                        wait_idx(nxt)
                        issue_gather(nxt[0].at[0], 0)
            else:
                issue_gather(cur[0].at[k + 1], (k + 1) % 2)
            wait_gather(cur[0].at[k], p)
            if k >= 2:
                wait_scatter(cur[1].at[k - 2], p)
            else:
                if is_a:
                    @pl.when(m >= 1)
                    def _():
                        wait_scatter(nxt[1].at[k + SCROWS - 2], p)
                else:
                    wait_scatter(nxt[1].at[k + SCROWS - 2], p)
            compute(cur, k, p)
            issue_scatter(cur[1].at[k], p)

    issue_pref(0, sets[0])
    wait_idx(sets[0])
    issue_gather(sets[0][0].at[0], 0)

    def mbody(m, carry):
        do_super(m, sets[0], sets[1], True)
        do_super(m, sets[1], sets[0], False)
        return carry
    lax.fori_loop(0, NM, mbody, 0)

    wait_scatter(sets[1][1].at[SCROWS - 2], 0)
    wait_scatter(sets[1][1].at[SCROWS - 1], 1)
    plsc.subcore_barrier()
    pltpu.sync_copy(acc.at[pl.ds(r0, RPT)], acc_hbm.at[c, pl.ds(r0, RPT)])


def _tc_norm_body(acc_ref, exp_ref, bias_ref, out_ref):
    e = exp_ref[:]
    halves = []
    for i in range(2):
        a = acc_ref[i]
        num = a[:, :64]
        den = a[:, 64:68]
        denr = jnp.dot(den, e, preferred_element_type=jnp.float32)
        halves.append(num / (denr + 1e-16))
    out_ref[:] = jnp.concatenate(halves, axis=1) + bias_ref[:]


def kernel(in_feat, edge_ind, edge_len, W, s_src, s_tgt, bias):
    in_feat = in_feat.astype(jnp.float32)
    x = jnp.pad(in_feat, ((0, NP - N), (0, 0)))
    Wt = W.astype(jnp.float32).T
    sf = s_src.reshape(-1).astype(jnp.float32)
    tf = s_tgt.reshape(-1).astype(jnp.float32)
    hsel = (jnp.arange(DIN)[:, None] // D) == jnp.arange(H)[None, :]
    Ssrc = jnp.where(hsel, sf[:, None], 0.0).T
    Stgt = jnp.where(hsel, tf[:, None], 0.0).T
    edges = edge_ind.astype(jnp.int32)
    pad_ix = jnp.full((EP - E,), NP - 1, jnp.int32)
    src = jnp.concatenate([edges[0], pad_ix])
    tgt = jnp.concatenate([edges[1], pad_ix])
    src2 = src.reshape(EP // CHB, CHB)
    tgt2 = tgt.reshape(EP // CHB, CHB)

    score_shapes = [jax.ShapeDtypeStruct((NP,), jnp.float32)] * (2 * H)
    outs = pl.pallas_call(
        _tc_proj_body,
        grid=(NBLK,),
        in_specs=[
            pl.BlockSpec((BLK, DIN), lambda i: (i, 0)),
            pl.BlockSpec((DIN, DIN), lambda i: (0, 0)),
            pl.BlockSpec((H, DIN), lambda i: (0, 0)),
            pl.BlockSpec((H, DIN), lambda i: (0, 0)),
        ],
        out_specs=[pl.BlockSpec((BLK, DIN), lambda i: (i, 0))]
        + [pl.BlockSpec((BLK,), lambda i: (i,))] * (2 * H),
        out_shape=[jax.ShapeDtypeStruct((NP, DIN), jnp.float32)] + score_shapes,
    )(x, Wt, Ssrc, Stgt)
    pe, scores = outs[0], outs[1:]

    mesh = plsc.VectorSubcoreMesh(core_axis_name="c", subcore_axis_name="s")
    sc_params = pltpu.CompilerParams(needs_layout_passes=False)

    score_k = functools.partial(
        pl.kernel,
        out_type=[jax.ShapeDtypeStruct((EP * HH,), jnp.float32)] * 2,
        mesh=mesh,
        compiler_params=sc_params,
        scratch_types=[
            pltpu.VMEM((HH * NP,), jnp.float32),
            pltpu.VMEM((HH * NP,), jnp.float32),
            pltpu.VMEM((CHA,), jnp.int32),
            pltpu.VMEM((CHA,), jnp.int32),
            pltpu.VMEM((CHA * HH,), jnp.float32),
        ],
    )(_sc_score_body)
    w0, w1 = score_k(src, tgt, *scores)

    agg_k = functools.partial(
        pl.kernel,
        out_type=jax.ShapeDtypeStruct((2, NP, RW), jnp.float32),
        mesh=mesh,
        compiler_params=sc_params,
        scratch_types=[
            pltpu.VMEM_SHARED((NP, RW), jnp.float32),
            pltpu.VMEM((SCROWS, CHB), jnp.int32),
            pltpu.VMEM((SCROWS, CHB), jnp.int32),
            pltpu.VMEM((SCE * HH + 16,), jnp.float32),
            pltpu.VMEM((SCROWS, CHB), jnp.int32),
            pltpu.VMEM((SCROWS, CHB), jnp.int32),
            pltpu.VMEM((SCE * HH + 16,), jnp.float32),
            pltpu.VMEM((CHB, RW), jnp.float32),
            pltpu.VMEM((CHB, RW), jnp.float32),
            pltpu.VMEM((CHB, RW), jnp.float32),
            pltpu.VMEM((CHB, RW), jnp.float32),
            pltpu.SemaphoreType.DMA,
            pltpu.SemaphoreType.DMA,
            pltpu.SemaphoreType.DMA,
            pltpu.SemaphoreType.DMA,
            pltpu.SemaphoreType.DMA,
            pltpu.SemaphoreType.DMA,
            pltpu.SemaphoreType.DMA,
            pltpu.SemaphoreType.DMA,
        ],
    )(_sc_agg_body)
    acc = agg_k(pe, w0, w1, src2, tgt2)

    expand = (jnp.arange(HH)[:, None] == (jnp.arange(64)[None, :] // D)
              ).astype(jnp.float32)

    out = pl.pallas_call(
        _tc_norm_body,
        grid=(NBLK,),
        in_specs=[
            pl.BlockSpec((2, BLK, RW), lambda i: (0, i, 0)),
            pl.BlockSpec((HH, 64), lambda i: (0, 0)),
            pl.BlockSpec((1, DIN), lambda i: (0, 0)),
        ],
        out_specs=pl.BlockSpec((BLK, DIN), lambda i: (i, 0)),
        out_shape=jax.ShapeDtypeStruct((NP, DIN), jnp.float32),
    )(acc, expand, bias.astype(jnp.float32).reshape(1, DIN))
    return out[:N]


# speedup vs baseline: 1.5163x; 1.5163x over previous
import functools

import jax
import jax.numpy as jnp
from jax import lax
from jax.experimental import pallas as pl
from jax.experimental.pallas import tpu as pltpu
from jax.experimental.pallas import tpu_sc as plsc

N = 10000
NP = 10240
E = 320000
EP = 327680
H = 8
D = 16
DIN = 128
RW = 128
HH = 4
NT = 16
RPT = NP // NT
EPT = EP // NT
CHA = 256
NCHA = EPT // CHA
DR = NP * HH // RW
DRT = DR // NT
DRH = DRT // 4
EPT2 = EP // (2 * NT)
CHB = 64
SCE = 512
SCROWS = SCE // CHB
NM = EPT2 // SCE // 2
BLK = 512
NBLK = NP // BLK


def _tc_proj_body(x_ref, wt_ref, ssrc_ref, stgt_ref, pe_ref, *score_refs):
    x = x_ref[:]
    p = jnp.dot(x, wt_ref[:], preferred_element_type=jnp.float32)
    ss = lax.dot_general(ssrc_ref[:], p, (((1,), (1,)), ((), ())),
                         preferred_element_type=jnp.float32)
    ts = lax.dot_general(stgt_ref[:], p, (((1,), (1,)), ((), ())),
                         preferred_element_type=jnp.float32)
    pe_ref[...] = p
    for g in range(H):
        score_refs[g][...] = ss[g]
        score_refs[H + g][...] = ts[g]


def _sc_score_body(src_hbm, tgt_hbm, *rest):
    score_hbm = rest[:2 * H]
    w0_hbm, w1_hbm, den_hbm = rest[2 * H:2 * H + 3]
    (stab, ttab, dtab, dacc, srcidx, tgtidx, wout, idb128, idb64,
     dbuf, oden) = rest[2 * H + 3:]
    c = lax.axis_index("c")
    t = lax.axis_index("s")
    iota16 = lax.iota(jnp.int32, 16)
    zeros16 = jnp.zeros((16,), jnp.float32)

    @pl.when(c == 0)
    def _():
        for h in range(HH):
            pltpu.sync_copy(score_hbm[h], stab.at[pl.ds(h * NP, NP)])
            pltpu.sync_copy(score_hbm[H + h], ttab.at[pl.ds(h * NP, NP)])

    @pl.when(c == 1)
    def _():
        for h in range(HH):
            pltpu.sync_copy(score_hbm[HH + h], stab.at[pl.ds(h * NP, NP)])
            pltpu.sync_copy(score_hbm[H + HH + h], ttab.at[pl.ds(h * NP, NP)])

    def zdrow(r, carry):
        for k in range(RW // 16):
            dtab[r, pl.ds(k * 16, 16)] = zeros16
        return carry
    lax.fori_loop(0, DR, zdrow, 0)
    pltpu.sync_copy(dtab.at[pl.ds(0, DRT)], dacc.at[pl.ds(t * DRT, DRT)])

    e0 = t * EPT

    def chunk(i, carry):
        base = e0 + i * CHA
        pltpu.sync_copy(src_hbm.at[pl.ds(base, CHA)], srcidx)
        pltpu.sync_copy(tgt_hbm.at[pl.ds(base, CHA)], tgtidx)
        for v in range(CHA // 16):
            sv = srcidx[pl.ds(v * 16, 16)]
            tv = tgtidx[pl.ds(v * 16, 16)]
            for h in range(HH):
                ssv = plsc.load_gather(stab, [sv + h * NP])
                tsv = plsc.load_gather(ttab, [tv + h * NP])
                s = ssv + tsv
                w = jnp.exp(jnp.where(s >= 0, s, s * 0.2))
                plsc.store_scatter(wout, [(iota16 + v * 16) * HH + h], w)
                p = tv * HH + h
                plsc.addupdate_scatter(
                    dtab, [lax.shift_right_logical(p, 7),
                           lax.bitwise_and(p, 127)], w)

        @pl.when(c == 0)
        def _():
            pltpu.sync_copy(wout, w0_hbm.at[pl.ds(base * HH, CHA * HH)])

        @pl.when(c == 1)
        def _():
            pltpu.sync_copy(wout, w1_hbm.at[pl.ds(base * HH, CHA * HH)])
        return carry

    lax.fori_loop(0, NCHA, chunk, 0)
    plsc.subcore_barrier()

    for k in range(2):
        def ibody(g, carry):
            plsc.store_scatter(idb128, [iota16 + g * 16],
                               iota16 + (g * 16 + k * 128))
            return carry
        lax.fori_loop(0, 8, ibody, 0)
        pltpu.sync_copy(dtab.at[pl.ds(k * 128, 128)], dacc.at[idb128], add=True)

    def ibody64(g, carry):
        plsc.store_scatter(idb64, [iota16 + g * 16], iota16 + (g * 16 + 256))
        return carry
    lax.fori_loop(0, 4, ibody64, 0)
    pltpu.sync_copy(dtab.at[pl.ds(256, DR - 256)], dacc.at[idb64], add=True)
    plsc.subcore_barrier()

    for hh2 in range(4):
        pltpu.sync_copy(dacc.at[pl.ds(t * DRT + hh2 * DRH, DRH)], dbuf)

        def rbody(r, carry):
            for s in range(8):
                lv = dbuf[r, pl.ds(s * 16, 16)]
                orow = (s % 2) * 4
                for u in range(4):
                    for h in range(HH):
                        oden[orow + u, pl.ds(h * D, D)] = jnp.full(
                            (D,), lv[u * HH + h], jnp.float32)
                if s % 2 == 1:
                    nb = t * RPT + hh2 * (RPT // 4) + r * 32 + (s // 2) * 8
                    pltpu.sync_copy(oden, den_hbm.at[c, pl.ds(nb, 8)])
            return carry
        lax.fori_loop(0, DRH, rbody, 0)


def _sc_agg_body(pe_hbm, w0_hbm, w1_hbm, src2_hbm, tgt2_hbm, acc_hbm,
                 acc, srcA, tgtA, w0A, w1A, srcB, tgtB, w0B, w1B,
                 rb0, rb1, ob0, ob1,
                 gs0, gs1, ss0, ss1, isA, isB, wsA, wsB):
    c = lax.axis_index("c")
    t = lax.axis_index("s")
    r0 = t * RPT
    row0 = (c * NT + t) * (EPT2 // CHB)
    e0 = (c * NT + t) * EPT2
    iota16 = lax.iota(jnp.int32, 16)
    zeros16 = jnp.zeros((16,), jnp.float32)
    rbs = (rb0, rb1)
    obs = (ob0, ob1)
    gss = (gs0, gs1)
    sss = (ss0, ss1)
    sets = ((srcA, tgtA, w0A, w1A, isA, wsA), (srcB, tgtB, w0B, w1B, isB, wsB))

    def zrow(j, carry):
        for k in range(RW // 16):
            ob0[j, pl.ds(k * 16, 16)] = zeros16
        return carry
    lax.fori_loop(0, CHB, zrow, 0)
    for k in range(RPT // CHB):
        pltpu.sync_copy(ob0, acc.at[pl.ds(r0 + k * CHB, CHB)])
    plsc.subcore_barrier()

    def issue_pref(g, bufset):
        sbuf, tbuf, w0b, w1b, isem, wsem = bufset
        rr = row0 + g * SCROWS
        pltpu.async_copy(src2_hbm.at[pl.ds(rr, SCROWS)], sbuf, isem)
        pltpu.async_copy(tgt2_hbm.at[pl.ds(rr, SCROWS)], tbuf, isem)
        eb = (e0 + g * SCE) * HH
        pltpu.async_copy(w0_hbm.at[pl.ds(eb, SCE * HH)],
                         w0b.at[pl.ds(0, SCE * HH)], wsem)
        pltpu.async_copy(w1_hbm.at[pl.ds(eb, SCE * HH)],
                         w1b.at[pl.ds(0, SCE * HH)], wsem)

    def wait_idx(bufset):
        sbuf, tbuf, w0b, w1b, isem, wsem = bufset
        pltpu.make_async_copy(src2_hbm.at[pl.ds(row0, SCROWS)], sbuf, isem).wait()
        pltpu.make_async_copy(tgt2_hbm.at[pl.ds(row0, SCROWS)], tbuf, isem).wait()

    def wait_w(bufset):
        sbuf, tbuf, w0b, w1b, isem, wsem = bufset
        pltpu.make_async_copy(w0_hbm.at[pl.ds(0, SCE * HH)],
                              w0b.at[pl.ds(0, SCE * HH)], wsem).wait()
        pltpu.make_async_copy(w1_hbm.at[pl.ds(0, SCE * HH)],
                              w1b.at[pl.ds(0, SCE * HH)], wsem).wait()

    def issue_gather(srcref, p):
        pltpu.async_copy(pe_hbm.at[srcref], rbs[p], gss[p])

    def wait_gather(srcref, p):
        pltpu.make_async_copy(pe_hbm.at[srcref], rbs[p], gss[p]).wait()

    def issue_scatter(tgtref, p):
        pltpu.async_copy(obs[p], acc.at[tgtref], sss[p], add=True)

    def wait_scatter(tgtref, p):
        pltpu.make_async_copy(obs[p], acc.at[tgtref], sss[p]).wait()

    def compute(bufset, k, p):
        sbuf, tbuf, w0b, w1b, isem, wsem = bufset
        ob = obs[p]
        rb = rbs[p]

        def jbody(j, carry):
            wb = k * CHB * HH + j * HH
            wv0 = w0b[pl.ds(wb, 16)]
            wv1 = w1b[pl.ds(wb, 16)]
            for h in range(H):
                w = wv0[h] if h < HH else wv1[h - HH]
                r = rb[j, pl.ds(h * D, D)]
                ob[j, pl.ds(h * D, D)] = r * w
            return carry
        lax.fori_loop(0, CHB, jbody, 0)

    def do_super(m, cur, nxt, is_a):
        wait_w(cur)
        for k in range(SCROWS):
            p = k % 2
            if k == 2:
                if is_a:
                    issue_pref(2 * m + 1, nxt)
                else:
                    @pl.when(m < NM - 1)
                    def _():
                        issue_pref(2 * m + 2, nxt)
            if k == SCROWS - 1:
                if is_a:
                    wait_idx(nxt)
                    issue_gather(nxt[0].at[0], 0)
                else:
                    @pl.when(m < NM - 1)
                    def _():
                        wait_idx(nxt)
                        issue_gather(nxt[0].at[0], 0)
            else:
                issue_gather(cur[0].at[k + 1], (k + 1) % 2)
            wait_gather(cur[0].at[k], p)
            if k >= 2:
                wait_scatter(cur[1].at[k - 2], p)
            else:
                if is_a:
                    @pl.when(m >= 1)
                    def _():
                        wait_scatter(nxt[1].at[k + SCROWS - 2], p)
                else:
                    wait_scatter(nxt[1].at[k + SCROWS - 2], p)
            compute(cur, k, p)
            issue_scatter(cur[1].at[k], p)

    issue_pref(0, sets[0])
    wait_idx(sets[0])
    issue_gather(sets[0][0].at[0], 0)

    def mbody(m, carry):
        do_super(m, sets[0], sets[1], True)
        do_super(m, sets[1], sets[0], False)
        return carry
    lax.fori_loop(0, NM, mbody, 0)

    wait_scatter(sets[1][1].at[SCROWS - 2], 0)
    wait_scatter(sets[1][1].at[SCROWS - 1], 1)
    plsc.subcore_barrier()
    pltpu.sync_copy(acc.at[pl.ds(r0, RPT)], acc_hbm.at[c, pl.ds(r0, RPT)])


def _tc_norm_body(acc_ref, den_ref, bias_ref, out_ref):
    numer = acc_ref[0] + acc_ref[1]
    denr = jnp.concatenate([den_ref[0], den_ref[1]], axis=1)
    out_ref[:] = numer / (denr + 1e-16) + bias_ref[:]


def kernel(in_feat, edge_ind, edge_len, W, s_src, s_tgt, bias):
    in_feat = in_feat.astype(jnp.float32)
    x = jnp.pad(in_feat, ((0, NP - N), (0, 0)))
    Wt = W.astype(jnp.float32).T
    sf = s_src.reshape(-1).astype(jnp.float32)
    tf = s_tgt.reshape(-1).astype(jnp.float32)
    hsel = (jnp.arange(DIN)[:, None] // D) == jnp.arange(H)[None, :]
    Ssrc = jnp.where(hsel, sf[:, None], 0.0).T
    Stgt = jnp.where(hsel, tf[:, None], 0.0).T
    edges = edge_ind.astype(jnp.int32)
    pad_ix = jnp.full((EP - E,), NP - 1, jnp.int32)
    src = jnp.concatenate([edges[0], pad_ix])
    tgt = jnp.concatenate([edges[1], pad_ix])
    src2 = src.reshape(EP // CHB, CHB)
    tgt2 = tgt.reshape(EP // CHB, CHB)

    score_shapes = [jax.ShapeDtypeStruct((NP,), jnp.float32)] * (2 * H)
    outs = pl.pallas_call(
        _tc_proj_body,
        grid=(NBLK,),
        in_specs=[
            pl.BlockSpec((BLK, DIN), lambda i: (i, 0)),
            pl.BlockSpec((DIN, DIN), lambda i: (0, 0)),
            pl.BlockSpec((H, DIN), lambda i: (0, 0)),
            pl.BlockSpec((H, DIN), lambda i: (0, 0)),
        ],
        out_specs=[pl.BlockSpec((BLK, DIN), lambda i: (i, 0))]
        + [pl.BlockSpec((BLK,), lambda i: (i,))] * (2 * H),
        out_shape=[jax.ShapeDtypeStruct((NP, DIN), jnp.float32)] + score_shapes,
    )(x, Wt, Ssrc, Stgt)
    pe, scores = outs[0], outs[1:]

    mesh = plsc.VectorSubcoreMesh(core_axis_name="c", subcore_axis_name="s")
    sc_params = pltpu.CompilerParams(needs_layout_passes=False)

    score_k = functools.partial(
        pl.kernel,
        out_type=[jax.ShapeDtypeStruct((EP * HH,), jnp.float32),
                  jax.ShapeDtypeStruct((EP * HH,), jnp.float32),
                  jax.ShapeDtypeStruct((2, NP, 64), jnp.float32)],
        mesh=mesh,
        compiler_params=sc_params,
        scratch_types=[
            pltpu.VMEM((HH * NP,), jnp.float32),
            pltpu.VMEM((HH * NP,), jnp.float32),
            pltpu.VMEM((DR, RW), jnp.float32),
            pltpu.VMEM_SHARED((DR, RW), jnp.float32),
            pltpu.VMEM((CHA,), jnp.int32),
            pltpu.VMEM((CHA,), jnp.int32),
            pltpu.VMEM((CHA * HH,), jnp.float32),
            pltpu.VMEM((128,), jnp.int32),
            pltpu.VMEM((64,), jnp.int32),
            pltpu.VMEM((DRH, RW), jnp.float32),
            pltpu.VMEM((8, 64), jnp.float32),
        ],
    )(_sc_score_body)
    w0, w1, den = score_k(src, tgt, *scores)

    agg_k = functools.partial(
        pl.kernel,
        out_type=jax.ShapeDtypeStruct((2, NP, RW), jnp.float32),
        mesh=mesh,
        compiler_params=sc_params,
        scratch_types=[
            pltpu.VMEM_SHARED((NP, RW), jnp.float32),
            pltpu.VMEM((SCROWS, CHB), jnp.int32),
            pltpu.VMEM((SCROWS, CHB), jnp.int32),
            pltpu.VMEM((SCE * HH + 16,), jnp.float32),
            pltpu.VMEM((SCE * HH + 16,), jnp.float32),
            pltpu.VMEM((SCROWS, CHB), jnp.int32),
            pltpu.VMEM((SCROWS, CHB), jnp.int32),
            pltpu.VMEM((SCE * HH + 16,), jnp.float32),
            pltpu.VMEM((SCE * HH + 16,), jnp.float32),
            pltpu.VMEM((CHB, RW), jnp.float32),
            pltpu.VMEM((CHB, RW), jnp.float32),
            pltpu.VMEM((CHB, RW), jnp.float32),
            pltpu.VMEM((CHB, RW), jnp.float32),
            pltpu.SemaphoreType.DMA,
            pltpu.SemaphoreType.DMA,
            pltpu.SemaphoreType.DMA,
            pltpu.SemaphoreType.DMA,
            pltpu.SemaphoreType.DMA,
            pltpu.SemaphoreType.DMA,
            pltpu.SemaphoreType.DMA,
            pltpu.SemaphoreType.DMA,
        ],
    )(_sc_agg_body)
    acc = agg_k(pe, w0, w1, src2, tgt2)

    out = pl.pallas_call(
        _tc_norm_body,
        grid=(NBLK,),
        in_specs=[
            pl.BlockSpec((2, BLK, RW), lambda i: (0, i, 0)),
            pl.BlockSpec((2, BLK, 64), lambda i: (0, i, 0)),
            pl.BlockSpec((1, DIN), lambda i: (0, 0)),
        ],
        out_specs=pl.BlockSpec((BLK, DIN), lambda i: (i, 0)),
        out_shape=jax.ShapeDtypeStruct((NP, DIN), jnp.float32),
    )(acc, den, bias.astype(jnp.float32).reshape(1, DIN))
    return out[:N]


# speedup vs baseline: 1.6854x; 1.1115x over previous
import functools

import jax
import jax.numpy as jnp
from jax import lax
from jax.experimental import pallas as pl
from jax.experimental.pallas import tpu as pltpu
from jax.experimental.pallas import tpu_sc as plsc

N = 10000
NP = 10240
E = 320000
EP = 327680
H = 8
D = 16
DIN = 128
RW = 128
HH = 4
NT = 16
RPT = NP // NT
EPT = EP // NT
CHA = 128
NCHA = EPT // CHA
NMA = NCHA // 2
DR = NP * HH // RW
DRT = DR // NT
DRH = DRT // 4
EPT2 = EP // (2 * NT)
CHB = 64
SCE = 512
SCROWS = SCE // CHB
NM = EPT2 // SCE // 2
BLK = 512
NBLK = NP // BLK


def _tc_proj_body(x_ref, wt_ref, ssrc_ref, stgt_ref, pe_ref, *score_refs):
    x = x_ref[:]
    p = jnp.dot(x, wt_ref[:], preferred_element_type=jnp.float32)
    ss = lax.dot_general(ssrc_ref[:], p, (((1,), (1,)), ((), ())),
                         preferred_element_type=jnp.float32)
    ts = lax.dot_general(stgt_ref[:], p, (((1,), (1,)), ((), ())),
                         preferred_element_type=jnp.float32)
    pe_ref[...] = p
    for g in range(H):
        score_refs[g][...] = ss[g]
        score_refs[H + g][...] = ts[g]


def _sc_score_body(src_hbm, tgt_hbm, *rest):
    score_hbm = rest[:2 * H]
    w0_hbm, w1_hbm, den_hbm = rest[2 * H:2 * H + 3]
    (stab, ttab, dtab, dacc, si0, ti0, wo0, si1, ti1, wo1, idb128, idb64,
     dbuf, oden, es0, es1, ws0, ws1) = rest[2 * H + 3:]
    c = lax.axis_index("c")
    t = lax.axis_index("s")
    iota16 = lax.iota(jnp.int32, 16)
    zeros16 = jnp.zeros((16,), jnp.float32)

    @pl.when(c == 0)
    def _():
        for h in range(HH):
            pltpu.sync_copy(score_hbm[h], stab.at[pl.ds(h * NP, NP)])
            pltpu.sync_copy(score_hbm[H + h], ttab.at[pl.ds(h * NP, NP)])

    @pl.when(c == 1)
    def _():
        for h in range(HH):
            pltpu.sync_copy(score_hbm[HH + h], stab.at[pl.ds(h * NP, NP)])
            pltpu.sync_copy(score_hbm[H + HH + h], ttab.at[pl.ds(h * NP, NP)])

    def zdrow(r, carry):
        for k in range(RW // 16):
            dtab[r, pl.ds(k * 16, 16)] = zeros16
        return carry
    lax.fori_loop(0, DR, zdrow, 0)
    pltpu.sync_copy(dtab.at[pl.ds(0, DRT)], dacc.at[pl.ds(t * DRT, DRT)])

    e0 = t * EPT
    esets = ((si0, ti0, wo0, es0, ws0), (si1, ti1, wo1, es1, ws1))

    def issue_idx(i, eset):
        si, ti, wo, esem, wsem = eset
        base = e0 + i * CHA
        pltpu.async_copy(src_hbm.at[pl.ds(base, CHA)], si, esem)
        pltpu.async_copy(tgt_hbm.at[pl.ds(base, CHA)], ti, esem)

    def wait_idx_e(eset):
        si, ti, wo, esem, wsem = eset
        pltpu.make_async_copy(src_hbm.at[pl.ds(0, CHA)], si, esem).wait()
        pltpu.make_async_copy(tgt_hbm.at[pl.ds(0, CHA)], ti, esem).wait()

    def issue_wb(i, eset):
        si, ti, wo, esem, wsem = eset
        base = e0 + i * CHA

        @pl.when(c == 0)
        def _():
            pltpu.async_copy(wo, w0_hbm.at[pl.ds(base * HH, CHA * HH)], wsem)

        @pl.when(c == 1)
        def _():
            pltpu.async_copy(wo, w1_hbm.at[pl.ds(base * HH, CHA * HH)], wsem)

    def wait_wb(eset):
        si, ti, wo, esem, wsem = eset

        @pl.when(c == 0)
        def _():
            pltpu.make_async_copy(
                wo, w0_hbm.at[pl.ds(0, CHA * HH)], wsem).wait()

        @pl.when(c == 1)
        def _():
            pltpu.make_async_copy(
                wo, w1_hbm.at[pl.ds(0, CHA * HH)], wsem).wait()

    def score_chunk(i, m2, eset):
        si, ti, wo, esem, wsem = eset
        wait_idx_e(eset)

        @pl.when(m2 >= 1)
        def _():
            wait_wb(eset)
        for v in range(CHA // 16):
            sv = si[pl.ds(v * 16, 16)]
            tv = ti[pl.ds(v * 16, 16)]
            for h in range(HH):
                ssv = plsc.load_gather(stab, [sv + h * NP])
                tsv = plsc.load_gather(ttab, [tv + h * NP])
                s = ssv + tsv
                w = jnp.exp(jnp.where(s >= 0, s, s * 0.2))
                plsc.store_scatter(wo, [(iota16 + v * 16) * HH + h], w)
                p = tv * HH + h
                plsc.addupdate_scatter(
                    dtab, [lax.shift_right_logical(p, 7),
                           lax.bitwise_and(p, 127)], w)
        issue_wb(i, eset)

    issue_idx(0, esets[0])

    def mbody2(m2, carry):
        issue_idx(2 * m2 + 1, esets[1])
        score_chunk(2 * m2, m2, esets[0])

        @pl.when(m2 < NMA - 1)
        def _():
            issue_idx(2 * m2 + 2, esets[0])
        score_chunk(2 * m2 + 1, m2, esets[1])
        return carry
    lax.fori_loop(0, NMA, mbody2, 0)
    wait_wb(esets[0])
    wait_wb(esets[1])
    plsc.subcore_barrier()

    for k in range(2):
        def ibody(g, carry):
            plsc.store_scatter(idb128, [iota16 + g * 16],
                               iota16 + (g * 16 + k * 128))
            return carry
        lax.fori_loop(0, 8, ibody, 0)
        pltpu.sync_copy(dtab.at[pl.ds(k * 128, 128)], dacc.at[idb128], add=True)

    def ibody64(g, carry):
        plsc.store_scatter(idb64, [iota16 + g * 16], iota16 + (g * 16 + 256))
        return carry
    lax.fori_loop(0, 4, ibody64, 0)
    pltpu.sync_copy(dtab.at[pl.ds(256, DR - 256)], dacc.at[idb64], add=True)
    plsc.subcore_barrier()

    for hh2 in range(4):
        pltpu.sync_copy(dacc.at[pl.ds(t * DRT + hh2 * DRH, DRH)], dbuf)

        def rbody(r, carry):
            for s in range(8):
                lv = dbuf[r, pl.ds(s * 16, 16)]
                orow = (s % 2) * 4
                for u in range(4):
                    for h in range(HH):
                        oden[orow + u, pl.ds(h * D, D)] = jnp.full(
                            (D,), lv[u * HH + h], jnp.float32)
                if s % 2 == 1:
                    nb = t * RPT + hh2 * (RPT // 4) + r * 32 + (s // 2) * 8
                    pltpu.sync_copy(oden, den_hbm.at[c, pl.ds(nb, 8)])
            return carry
        lax.fori_loop(0, DRH, rbody, 0)


def _sc_agg_body(pe_hbm, w0_hbm, w1_hbm, src2_hbm, tgt2_hbm, acc_hbm,
                 acc, srcA, tgtA, w0A, w1A, srcB, tgtB, w0B, w1B,
                 rb0, rb1, ob0, ob1,
                 gs0, gs1, ss0, ss1, isA, isB, wsA, wsB):
    c = lax.axis_index("c")
    t = lax.axis_index("s")
    r0 = t * RPT
    row0 = (c * NT + t) * (EPT2 // CHB)
    e0 = (c * NT + t) * EPT2
    iota16 = lax.iota(jnp.int32, 16)
    zeros16 = jnp.zeros((16,), jnp.float32)
    rbs = (rb0, rb1)
    obs = (ob0, ob1)
    gss = (gs0, gs1)
    sss = (ss0, ss1)
    sets = ((srcA, tgtA, w0A, w1A, isA, wsA), (srcB, tgtB, w0B, w1B, isB, wsB))

    def zrow(j, carry):
        for k in range(RW // 16):
            ob0[j, pl.ds(k * 16, 16)] = zeros16
        return carry
    lax.fori_loop(0, CHB, zrow, 0)
    for k in range(RPT // CHB):
        pltpu.sync_copy(ob0, acc.at[pl.ds(r0 + k * CHB, CHB)])
    plsc.subcore_barrier()

    def issue_pref(g, bufset):
        sbuf, tbuf, w0b, w1b, isem, wsem = bufset
        rr = row0 + g * SCROWS
        pltpu.async_copy(src2_hbm.at[pl.ds(rr, SCROWS)], sbuf, isem)
        pltpu.async_copy(tgt2_hbm.at[pl.ds(rr, SCROWS)], tbuf, isem)
        eb = (e0 + g * SCE) * HH
        pltpu.async_copy(w0_hbm.at[pl.ds(eb, SCE * HH)],
                         w0b.at[pl.ds(0, SCE * HH)], wsem)
        pltpu.async_copy(w1_hbm.at[pl.ds(eb, SCE * HH)],
                         w1b.at[pl.ds(0, SCE * HH)], wsem)

    def wait_idx(bufset):
        sbuf, tbuf, w0b, w1b, isem, wsem = bufset
        pltpu.make_async_copy(src2_hbm.at[pl.ds(row0, SCROWS)], sbuf, isem).wait()
        pltpu.make_async_copy(tgt2_hbm.at[pl.ds(row0, SCROWS)], tbuf, isem).wait()

    def wait_w(bufset):
        sbuf, tbuf, w0b, w1b, isem, wsem = bufset
        pltpu.make_async_copy(w0_hbm.at[pl.ds(0, SCE * HH)],
                              w0b.at[pl.ds(0, SCE * HH)], wsem).wait()
        pltpu.make_async_copy(w1_hbm.at[pl.ds(0, SCE * HH)],
                              w1b.at[pl.ds(0, SCE * HH)], wsem).wait()

    def issue_gather(srcref, p):
        pltpu.async_copy(pe_hbm.at[srcref], rbs[p], gss[p])

    def wait_gather(srcref, p):
        pltpu.make_async_copy(pe_hbm.at[srcref], rbs[p], gss[p]).wait()

    def issue_scatter(tgtref, p):
        pltpu.async_copy(obs[p], acc.at[tgtref], sss[p], add=True)

    def wait_scatter(tgtref, p):
        pltpu.make_async_copy(obs[p], acc.at[tgtref], sss[p]).wait()

    def compute(bufset, k, p):
        sbuf, tbuf, w0b, w1b, isem, wsem = bufset
        ob = obs[p]
        rb = rbs[p]

        def jbody(j, carry):
            wb = k * CHB * HH + j * HH
            wv0 = w0b[pl.ds(wb, 16)]
            wv1 = w1b[pl.ds(wb, 16)]
            for h in range(H):
                w = wv0[h] if h < HH else wv1[h - HH]
                r = rb[j, pl.ds(h * D, D)]
                ob[j, pl.ds(h * D, D)] = r * w
            return carry
        lax.fori_loop(0, CHB, jbody, 0)

    def do_super(m, cur, nxt, is_a):
        wait_w(cur)
        for k in range(SCROWS):
            p = k % 2
            if k == 2:
                if is_a:
                    issue_pref(2 * m + 1, nxt)
                else:
                    @pl.when(m < NM - 1)
                    def _():
                        issue_pref(2 * m + 2, nxt)
            if k == SCROWS - 1:
                if is_a:
                    wait_idx(nxt)
                    issue_gather(nxt[0].at[0], 0)
                else:
                    @pl.when(m < NM - 1)
                    def _():
                        wait_idx(nxt)
                        issue_gather(nxt[0].at[0], 0)
            else:
                issue_gather(cur[0].at[k + 1], (k + 1) % 2)
            wait_gather(cur[0].at[k], p)
            if k >= 2:
                wait_scatter(cur[1].at[k - 2], p)
            else:
                if is_a:
                    @pl.when(m >= 1)
                    def _():
                        wait_scatter(nxt[1].at[k + SCROWS - 2], p)
                else:
                    wait_scatter(nxt[1].at[k + SCROWS - 2], p)
            compute(cur, k, p)
            issue_scatter(cur[1].at[k], p)

    issue_pref(0, sets[0])
    wait_idx(sets[0])
    issue_gather(sets[0][0].at[0], 0)

    def mbody(m, carry):
        do_super(m, sets[0], sets[1], True)
        do_super(m, sets[1], sets[0], False)
        return carry
    lax.fori_loop(0, NM, mbody, 0)

    wait_scatter(sets[1][1].at[SCROWS - 2], 0)
    wait_scatter(sets[1][1].at[SCROWS - 1], 1)
    plsc.subcore_barrier()
    pltpu.sync_copy(acc.at[pl.ds(r0, RPT)], acc_hbm.at[c, pl.ds(r0, RPT)])


def _tc_norm_body(acc_ref, den_ref, bias_ref, out_ref):
    numer = acc_ref[0] + acc_ref[1]
    denr = jnp.concatenate([den_ref[0], den_ref[1]], axis=1)
    out_ref[:] = numer / (denr + 1e-16) + bias_ref[:]


def kernel(in_feat, edge_ind, edge_len, W, s_src, s_tgt, bias):
    in_feat = in_feat.astype(jnp.float32)
    x = jnp.pad(in_feat, ((0, NP - N), (0, 0)))
    Wt = W.astype(jnp.float32).T
    sf = s_src.reshape(-1).astype(jnp.float32)
    tf = s_tgt.reshape(-1).astype(jnp.float32)
    hsel = (jnp.arange(DIN)[:, None] // D) == jnp.arange(H)[None, :]
    Ssrc = jnp.where(hsel, sf[:, None], 0.0).T
    Stgt = jnp.where(hsel, tf[:, None], 0.0).T
    edges = edge_ind.astype(jnp.int32)
    pad_ix = jnp.full((EP - E,), NP - 1, jnp.int32)
    src = jnp.concatenate([edges[0], pad_ix])
    tgt = jnp.concatenate([edges[1], pad_ix])
    src2 = src.reshape(EP // CHB, CHB)
    tgt2 = tgt.reshape(EP // CHB, CHB)

    score_shapes = [jax.ShapeDtypeStruct((NP,), jnp.float32)] * (2 * H)
    outs = pl.pallas_call(
        _tc_proj_body,
        grid=(NBLK,),
        in_specs=[
            pl.BlockSpec((BLK, DIN), lambda i: (i, 0)),
            pl.BlockSpec((DIN, DIN), lambda i: (0, 0)),
            pl.BlockSpec((H, DIN), lambda i: (0, 0)),
            pl.BlockSpec((H, DIN), lambda i: (0, 0)),
        ],
        out_specs=[pl.BlockSpec((BLK, DIN), lambda i: (i, 0))]
        + [pl.BlockSpec((BLK,), lambda i: (i,))] * (2 * H),
        out_shape=[jax.ShapeDtypeStruct((NP, DIN), jnp.float32)] + score_shapes,
    )(x, Wt, Ssrc, Stgt)
    pe, scores = outs[0], outs[1:]

    mesh = plsc.VectorSubcoreMesh(core_axis_name="c", subcore_axis_name="s")
    sc_params = pltpu.CompilerParams(needs_layout_passes=False)

    score_k = functools.partial(
        pl.kernel,
        out_type=[jax.ShapeDtypeStruct((EP * HH,), jnp.float32),
                  jax.ShapeDtypeStruct((EP * HH,), jnp.float32),
                  jax.ShapeDtypeStruct((2, NP, 64), jnp.float32)],
        mesh=mesh,
        compiler_params=sc_params,
        scratch_types=[
            pltpu.VMEM((HH * NP,), jnp.float32),
            pltpu.VMEM((HH * NP,), jnp.float32),
            pltpu.VMEM((DR, RW), jnp.float32),
            pltpu.VMEM_SHARED((DR, RW), jnp.float32),
            pltpu.VMEM((CHA,), jnp.int32),
            pltpu.VMEM((CHA,), jnp.int32),
            pltpu.VMEM((CHA * HH,), jnp.float32),
            pltpu.VMEM((CHA,), jnp.int32),
            pltpu.VMEM((CHA,), jnp.int32),
            pltpu.VMEM((CHA * HH,), jnp.float32),
            pltpu.VMEM((128,), jnp.int32),
            pltpu.VMEM((64,), jnp.int32),
            pltpu.VMEM((DRH, RW), jnp.float32),
            pltpu.VMEM((8, 64), jnp.float32),
            pltpu.SemaphoreType.DMA,
            pltpu.SemaphoreType.DMA,
            pltpu.SemaphoreType.DMA,
            pltpu.SemaphoreType.DMA,
        ],
    )(_sc_score_body)
    w0, w1, den = score_k(src, tgt, *scores)

    agg_k = functools.partial(
        pl.kernel,
        out_type=jax.ShapeDtypeStruct((2, NP, RW), jnp.float32),
        mesh=mesh,
        compiler_params=sc_params,
        scratch_types=[
            pltpu.VMEM_SHARED((NP, RW), jnp.float32),
            pltpu.VMEM((SCROWS, CHB), jnp.int32),
            pltpu.VMEM((SCROWS, CHB), jnp.int32),
            pltpu.VMEM((SCE * HH + 16,), jnp.float32),
            pltpu.VMEM((SCE * HH + 16,), jnp.float32),
            pltpu.VMEM((SCROWS, CHB), jnp.int32),
            pltpu.VMEM((SCROWS, CHB), jnp.int32),
            pltpu.VMEM((SCE * HH + 16,), jnp.float32),
            pltpu.VMEM((SCE * HH + 16,), jnp.float32),
            pltpu.VMEM((CHB, RW), jnp.float32),
            pltpu.VMEM((CHB, RW), jnp.float32),
            pltpu.VMEM((CHB, RW), jnp.float32),
            pltpu.VMEM((CHB, RW), jnp.float32),
            pltpu.SemaphoreType.DMA,
            pltpu.SemaphoreType.DMA,
            pltpu.SemaphoreType.DMA,
            pltpu.SemaphoreType.DMA,
            pltpu.SemaphoreType.DMA,
            pltpu.SemaphoreType.DMA,
            pltpu.SemaphoreType.DMA,
            pltpu.SemaphoreType.DMA,
        ],
    )(_sc_agg_body)
    acc = agg_k(pe, w0, w1, src2, tgt2)

    out = pl.pallas_call(
        _tc_norm_body,
        grid=(NBLK,),
        in_specs=[
            pl.BlockSpec((2, BLK, RW), lambda i: (0, i, 0)),
            pl.BlockSpec((2, BLK, 64), lambda i: (0, i, 0)),
            pl.BlockSpec((1, DIN), lambda i: (0, 0)),
        ],
        out_specs=pl.BlockSpec((BLK, DIN), lambda i: (i, 0)),
        out_shape=jax.ShapeDtypeStruct((NP, DIN), jnp.float32),
    )(acc, den, bias.astype(jnp.float32).reshape(1, DIN))
    return out[:N]
